# Initial kernel scaffold; baseline (speedup 1.0000x reference)
import jax
import jax.numpy as jnp
from jax.experimental import pallas as pl


def kernel(x, edge_index, batch, W_gat, att_src, att_dst, bias_gat, W1, b1, W2, b2, ln1_g, ln1_b, ln2_g, ln2_b, gate_W, gate_b):
    raise NotImplementedError("write your pallas kernel here")


# speedup vs baseline: 1.1448x; 1.1448x over previous
import functools
import jax
import jax.numpy as jnp
from jax import lax
from jax.experimental import pallas as pl
from jax.experimental.pallas import tpu as pltpu

_N = 10000
_E = 160000
_D = 256
_H = 8
_C = 256
_DFF = 512
_B = 32

_NEG = -3.4e38


def _prep_body(x_ref, wgat_ref, atts_ref, attd_ref,
               as_ref, ad_ref, ws_ref, wd_ref, m_ref):
    i = pl.program_id(0)
    cols_s = []
    cols_d = []
    for h in range(_H):
        wblk = wgat_ref[:, h * _C:(h + 1) * _C]
        cols_s.append(jnp.dot(wblk, atts_ref[h, :], preferred_element_type=jnp.float32))
        cols_d.append(jnp.dot(wblk, attd_ref[h, :], preferred_element_type=jnp.float32))
    ws = jnp.stack(cols_s, axis=1)
    wd = jnp.stack(cols_d, axis=1)
    ws_ref[...] = ws
    wd_ref[...] = wd
    a_s = jnp.dot(x_ref[...], ws, preferred_element_type=jnp.float32)
    a_d = jnp.dot(x_ref[...], wd, preferred_element_type=jnp.float32)
    as_ref[...] = a_s
    ad_ref[...] = a_d

    @pl.when(i == 0)
    def _():
        m_ref[...] = jnp.full_like(m_ref, _NEG)
    m_ref[...] = jnp.maximum(m_ref[...],
                             jnp.stack([a_s.max(axis=0), a_d.max(axis=0)]))


def _prep(x, W_gat, att_src, att_dst):
    blk = 2000
    grid = (_N // blk,)
    return pl.pallas_call(
        _prep_body,
        grid=grid,
        in_specs=[
            pl.BlockSpec((blk, _D), lambda i: (i, 0)),
            pl.BlockSpec((_D, _H * _C), lambda i: (0, 0)),
            pl.BlockSpec((_H, _C), lambda i: (0, 0)),
            pl.BlockSpec((_H, _C), lambda i: (0, 0)),
        ],
        out_specs=[
            pl.BlockSpec((blk, _H), lambda i: (i, 0)),
            pl.BlockSpec((blk, _H), lambda i: (i, 0)),
            pl.BlockSpec((_D, _H), lambda i: (0, 0)),
            pl.BlockSpec((_D, _H), lambda i: (0, 0)),
            pl.BlockSpec((2, _H), lambda i: (0, 0)),
        ],
        out_shape=[
            jax.ShapeDtypeStruct((_N, _H), jnp.float32),
            jax.ShapeDtypeStruct((_N, _H), jnp.float32),
            jax.ShapeDtypeStruct((_D, _H), jnp.float32),
            jax.ShapeDtypeStruct((_D, _H), jnp.float32),
            jax.ShapeDtypeStruct((2, _H), jnp.float32),
        ],
    )(x, W_gat, att_src, att_dst)


def _gatout_body(u_ref, wflat_ref, bias_ref, ws_ref, wd_ref,
                 out_ref, as_ref, ad_ref, m_ref):
    i = pl.program_id(0)
    out = jnp.dot(u_ref[...], wflat_ref[...], preferred_element_type=jnp.float32)
    out = out * (1.0 / _H) + bias_ref[...]
    out_ref[...] = out
    a_s = jnp.dot(out, ws_ref[...], preferred_element_type=jnp.float32)
    a_d = jnp.dot(out, wd_ref[...], preferred_element_type=jnp.float32)
    as_ref[...] = a_s
    ad_ref[...] = a_d

    @pl.when(i == 0)
    def _():
        m_ref[...] = jnp.full_like(m_ref, _NEG)
    m_ref[...] = jnp.maximum(m_ref[...],
                             jnp.stack([a_s.max(axis=0), a_d.max(axis=0)]))


def _gatout(U, Wflat, bias, Ws, Wd):
    blk = 1000
    grid = (_N // blk,)
    return pl.pallas_call(
        _gatout_body,
        grid=grid,
        in_specs=[
            pl.BlockSpec((blk, _H * _D), lambda i: (i, 0)),
            pl.BlockSpec((_H * _D, _C), lambda i: (0, 0)),
            pl.BlockSpec((1, _C), lambda i: (0, 0)),
            pl.BlockSpec((_D, _H), lambda i: (0, 0)),
            pl.BlockSpec((_D, _H), lambda i: (0, 0)),
        ],
        out_specs=[
            pl.BlockSpec((blk, _C), lambda i: (i, 0)),
            pl.BlockSpec((blk, _H), lambda i: (i, 0)),
            pl.BlockSpec((blk, _H), lambda i: (i, 0)),
            pl.BlockSpec((2, _H), lambda i: (0, 0)),
        ],
        out_shape=[
            jax.ShapeDtypeStruct((_N, _C), jnp.float32),
            jax.ShapeDtypeStruct((_N, _H), jnp.float32),
            jax.ShapeDtypeStruct((_N, _H), jnp.float32),
            jax.ShapeDtypeStruct((2, _H), jnp.float32),
        ],
    )(U, Wflat, bias, Ws, Wd)


def _gatout2_body(u_ref, wflat_ref, bias_ref, out_ref):
    out = jnp.dot(u_ref[...], wflat_ref[...], preferred_element_type=jnp.float32)
    out_ref[...] = out * (1.0 / _H) + bias_ref[...]


def _gatout2(U, Wflat, bias):
    blk = 1000
    return pl.pallas_call(
        _gatout2_body,
        grid=(_N // blk,),
        in_specs=[
            pl.BlockSpec((blk, _H * _D), lambda i: (i, 0)),
            pl.BlockSpec((_H * _D, _C), lambda i: (0, 0)),
            pl.BlockSpec((1, _C), lambda i: (0, 0)),
        ],
        out_specs=pl.BlockSpec((blk, _C), lambda i: (i, 0)),
        out_shape=jax.ShapeDtypeStruct((_N, _C), jnp.float32),
    )(U, Wflat, bias)


def _ffn_body(x1_ref, x2_ref, w1_ref, b1_ref, w2_ref, b2_ref,
              ln1g_ref, ln1b_ref, ln2g_ref, ln2b_ref, gw_ref, gb_ref,
              po_ref, gate_ref, gmax_ref):
    i = pl.program_id(0)
    s = x1_ref[...] + x2_ref[...]
    mu = s.mean(axis=-1, keepdims=True)
    var = ((s - mu) ** 2).mean(axis=-1, keepdims=True)
    pi = (s - mu) * lax.rsqrt(var + 1e-5) * ln1g_ref[...] + ln1b_ref[...]
    hdn = jnp.maximum(jnp.dot(pi, w1_ref[...], preferred_element_type=jnp.float32) + b1_ref[...], 0.0)
    ff = jnp.dot(hdn, w2_ref[...], preferred_element_type=jnp.float32) + b2_ref[...]
    t = pi + ff
    mu2 = t.mean(axis=-1, keepdims=True)
    var2 = ((t - mu2) ** 2).mean(axis=-1, keepdims=True)
    po = (t - mu2) * lax.rsqrt(var2 + 1e-5) * ln2g_ref[...] + ln2b_ref[...]
    po_ref[...] = po
    gate = jnp.dot(po, gw_ref[...], preferred_element_type=jnp.float32) + gb_ref[...]
    gate_ref[...] = gate

    @pl.when(i == 0)
    def _():
        gmax_ref[...] = jnp.full_like(gmax_ref, _NEG)
    gmax_ref[...] = jnp.maximum(gmax_ref[...], gate.max())


def _ffn(x1, x2, W1, b1, W2, b2, ln1g, ln1b, ln2g, ln2b, gW, gb):
    blk = 2000
    c0 = lambda i: (0, 0)
    return pl.pallas_call(
        _ffn_body,
        grid=(_N // blk,),
        in_specs=[
            pl.BlockSpec((blk, _C), lambda i: (i, 0)),
            pl.BlockSpec((blk, _C), lambda i: (i, 0)),
            pl.BlockSpec((_C, _DFF), c0),
            pl.BlockSpec((1, _DFF), c0),
            pl.BlockSpec((_DFF, _C), c0),
            pl.BlockSpec((1, _C), c0),
            pl.BlockSpec((1, _C), c0),
            pl.BlockSpec((1, _C), c0),
            pl.BlockSpec((1, _C), c0),
            pl.BlockSpec((1, _C), c0),
            pl.BlockSpec((_C, 1), c0),
            pl.BlockSpec((1, 1), c0),
        ],
        out_specs=[
            pl.BlockSpec((blk, _C), lambda i: (i, 0)),
            pl.BlockSpec((blk, 1), lambda i: (i, 0)),
            pl.BlockSpec((1, 1), c0),
        ],
        out_shape=[
            jax.ShapeDtypeStruct((_N, _C), jnp.float32),
            jax.ShapeDtypeStruct((_N, 1), jnp.float32),
            jax.ShapeDtypeStruct((1, 1), jnp.float32),
        ],
    )(x1, x2, W1, b1, W2, b2, ln1g, ln1b, ln2g, ln2b, gW, gb)


def _pool_body(po_ref, gate_ref, gmax_ref, batch_ref, out_ref, s_ref, den_ref):
    i = pl.program_id(0)
    nsteps = pl.num_programs(0)

    @pl.when(i == 0)
    def _():
        s_ref[...] = jnp.zeros_like(s_ref)
        den_ref[...] = jnp.zeros_like(den_ref)

    ex = jnp.exp(gate_ref[...] - gmax_ref[...])
    bvec = batch_ref[...]
    bid = jax.lax.broadcasted_iota(jnp.int32, (1, _B), 1)
    P = (bvec == bid).astype(jnp.float32)
    Pex = P * ex
    s_ref[...] += lax.dot_general(Pex, po_ref[...], (((0,), (0,)), ((), ())),
                                  preferred_element_type=jnp.float32)
    den_ref[...] += lax.dot_general(P, ex, (((0,), (0,)), ((), ())),
                                    preferred_element_type=jnp.float32)

    @pl.when(i == nsteps - 1)
    def _():
        out_ref[...] = s_ref[...] / (den_ref[...] + 1e-16)


def _pool(po, gate, gmax, batch2d):
    blk = 2000
    c0 = lambda i: (0, 0)
    return pl.pallas_call(
        _pool_body,
        grid=(_N // blk,),
        in_specs=[
            pl.BlockSpec((blk, _C), lambda i: (i, 0)),
            pl.BlockSpec((blk, 1), lambda i: (i, 0)),
            pl.BlockSpec((1, 1), c0),
            pl.BlockSpec((blk, 1), lambda i: (i, 0)),
        ],
        out_specs=pl.BlockSpec((_B, _C), c0),
        out_shape=jax.ShapeDtypeStruct((_B, _C), jnp.float32),
        scratch_shapes=[
            pltpu.VMEM((_B, _C), jnp.float32),
            pltpu.VMEM((_B, 1), jnp.float32),
        ],
    )(po, gate, gmax, batch2d)


def _edge_phase(x_in, a_s, a_d, M, src, dst):
    alpha = a_s[src] + a_d[dst]
    alpha = jnp.where(alpha >= 0.0, alpha, 0.2 * alpha)
    ex = jnp.exp(alpha - M[None, :])
    den = jax.ops.segment_sum(ex, dst, num_segments=_N)
    coef = ex / (den[dst] + 1e-16)
    U = jax.ops.segment_sum(coef[:, :, None] * x_in[src][:, None, :], dst,
                            num_segments=_N)
    return U.reshape(_N, _H * _D)


def kernel(x, edge_index, batch, W_gat, att_src, att_dst, bias_gat,
           W1, b1, W2, b2, ln1_g, ln1_b, ln2_g, ln2_b, gate_W, gate_b):
    loops = jnp.arange(_N, dtype=edge_index.dtype)
    src = jnp.concatenate([edge_index[0], loops])
    dst = jnp.concatenate([edge_index[1], loops])

    W3 = W_gat.reshape(_D, _H, _C)
    Wflat = W3.transpose(1, 0, 2).reshape(_H * _D, _C)
    bias2 = bias_gat.reshape(1, _C)

    a_s1, a_d1, Ws, Wd, m1 = _prep(x, W_gat, att_src, att_dst)
    M1 = m1[0] + m1[1]
    M1 = jnp.where(M1 >= 0, M1, 0.2 * M1)

    U1 = _edge_phase(x, a_s1, a_d1, M1, src, dst)
    x1, a_s2, a_d2, m2 = _gatout(U1, Wflat, bias2, Ws, Wd)
    M2 = m2[0] + m2[1]
    M2 = jnp.where(M2 >= 0, M2, 0.2 * M2)

    U2 = _edge_phase(x1, a_s2, a_d2, M2, src, dst)
    x2 = _gatout2(U2, Wflat, bias2)

    po, gate, gmax = _ffn(x1, x2, W1, b1.reshape(1, _DFF), W2, b2.reshape(1, _C),
                          ln1_g.reshape(1, _C), ln1_b.reshape(1, _C),
                          ln2_g.reshape(1, _C), ln2_b.reshape(1, _C),
                          gate_W, gate_b.reshape(1, 1))
    return _pool(po, gate, gmax, batch.reshape(_N, 1))
